# baseline (speedup 1.0000x reference)
import functools

import jax
import jax.numpy as jnp
from jax import lax
from jax.experimental import pallas as pl
from jax.experimental.pallas import tpu as pltpu
from jax.experimental.pallas import tpu_sc as plsc

_B, _F, _L, _V, _D = 4096, 26, 20, 100000, 32
_N = _B * _F
_NW = 32
_BATCH_PER_W = _B // _NW
_CB = 2
_CHUNKS = _BATCH_PER_W // _CB
_SEG = _CB * _F
_GATHER_W = _CB * _L
_IDX_PER_CHUNK = _SEG * _L
_NCHUNK_ROWS = _B * _F * _L // _IDX_PER_CHUNK


def kernel(indices, tables):
    idx = (
        indices.astype(jnp.int32)
        .reshape(_NCHUNK_ROWS, _CB, _F, _L)
        .transpose(0, 2, 1, 3)
        .reshape(_NCHUNK_ROWS, _IDX_PER_CHUNK)
    )

    mesh = plsc.VectorSubcoreMesh(core_axis_name="c", subcore_axis_name="s")

    @functools.partial(
        pl.kernel,
        mesh=mesh,
        compiler_params=pltpu.CompilerParams(use_tc_tiling_on_sc=False),
        out_type=jax.ShapeDtypeStruct((_N, _D), jnp.float32),
        scratch_types=[
            pltpu.VMEM((_IDX_PER_CHUNK,), jnp.int32),
            pltpu.VMEM((_IDX_PER_CHUNK,), jnp.int32),
            pltpu.VMEM((_IDX_PER_CHUNK, _D), jnp.float32),
            pltpu.VMEM((_IDX_PER_CHUNK, _D), jnp.float32),
            pltpu.VMEM((_SEG, _D), jnp.float32),
            pltpu.VMEM((_SEG, _D), jnp.float32),
            pltpu.SemaphoreType.DMA,
            pltpu.SemaphoreType.DMA,
        ],
    )
    def sc_kernel(tab_hbm, idx_hbm, out_hbm,
                  idx0, idx1, rows0, rows1, out0, out1, sem0, sem1):
        wid = lax.axis_index("s") * 2 + lax.axis_index("c")

        def fire(chunk, idx_v, rows_v, sem):
            crow = wid * _CHUNKS + chunk
            pltpu.sync_copy(idx_hbm.at[crow], idx_v)

            @pl.loop(0, _F)
            def _(f):
                sl = pl.ds(f * _GATHER_W, _GATHER_W)
                pltpu.async_copy(
                    tab_hbm.at[f].at[idx_v.at[sl]], rows_v.at[sl], sem
                )

        def drain(idx_v, rows_v, sem):
            @pl.loop(0, _F)
            def _(f):
                sl = pl.ds(f * _GATHER_W, _GATHER_W)
                pltpu.make_async_copy(
                    tab_hbm.at[f].at[idx_v.at[sl]], rows_v.at[sl], sem
                ).wait()

        def acc_store(chunk, rows_v, out_v):
            @pl.loop(0, _SEG)
            def _(r):
                base = r * _L
                f = r // _CB
                lb = r - f * _CB
                o = lb * _F + f
                for c in range(2):
                    csl = pl.ds(c * 16, 16)
                    acc_a = rows_v[base, csl]
                    acc_b = rows_v[base + 1, csl]
                    for l in range(2, _L, 2):
                        acc_a = acc_a + rows_v[base + l, csl]
                        acc_b = acc_b + rows_v[base + l + 1, csl]
                    out_v[o, csl] = acc_a + acc_b

            s_base = (wid * _CHUNKS + chunk) * _SEG
            pltpu.sync_copy(out_v, out_hbm.at[pl.ds(s_base, _SEG)])

        fire(0, idx0, rows0, sem0)

        @pl.loop(0, _CHUNKS // 2)
        def _(g):
            c0 = 2 * g
            c1 = c0 + 1
            c2 = jnp.where(c0 + 2 >= _CHUNKS, 0, c0 + 2)
            fire(c1, idx1, rows1, sem1)
            drain(idx0, rows0, sem0)
            acc_store(c0, rows0, out0)
            fire(c2, idx0, rows0, sem0)
            drain(idx1, rows1, sem1)
            acc_store(c1, rows1, out1)

        drain(idx0, rows0, sem0)

    return sc_kernel(tables, idx).reshape(_B, _F, _D)
